# baseline (speedup 1.0000x reference)
import functools

import jax
import jax.numpy as jnp
from jax import lax
from jax.experimental import pallas as pl
from jax.experimental.pallas import tpu as pltpu
from jax.experimental.pallas import tpu_sc as plsc

_N, _E, _NF, _EF, _H, _L = 10000, 320000, 128, 16, 128, 6
_NC, _NS = 2, 16
_HC = _H // _NC
_KE = 80
_NBT = _E // (_NS * _KE)
_GRP = 50
_NG = _NBT // _GRP
_EPT = _NBT * _KE
_RPT = 624
_REM = _N - _NS * _RPT
_ZR = 208

_BN = 400
_BE = 3200


def _sc_edge_body(with_deg, *refs):
    if with_deg:
        (a_hbm, b_hbm, c_hbm, src_hbm, dst_hbm, out_s, out_deg,
         idx_s, idx_d, idx_dg, a0, a1, b0, b1, r0, r1,
         ones_v, s_sh, deg_sh,
         sem_a0, sem_a1, sem_b0, sem_b1, sem_c0, sem_c1,
         sem_s0, sem_s1) = refs
    else:
        (a_hbm, b_hbm, c_hbm, src_hbm, dst_hbm, out_s,
         idx_s, idx_d, idx_dg, a0, a1, b0, b1, r0, r1, s_sh,
         sem_a0, sem_a1, sem_b0, sem_b1, sem_c0, sem_c1,
         sem_s0, sem_s1) = refs

    a_b = (a0, a1)
    b_b = (b0, b1)
    r_b = (r0, r1)
    sem_a = (sem_a0, sem_a1)
    sem_b = (sem_b0, sem_b1)
    sem_c = (sem_c0, sem_c1)
    sem_s = (sem_s0, sem_s1)

    cid = lax.axis_index("c")
    sid = lax.axis_index("s")
    do_deg = with_deg
    row_off = cid * _N

    def _zrow(k, carry):
        for cc in range(_HC // 16):
            sl = pl.ds(cc * 16, 16)
            r0[k, sl] = jnp.zeros((16,), jnp.float32)
        return carry

    lax.fori_loop(0, _KE, _zrow, 0)
    base = sid * _RPT
    for i in range(7):
        pltpu.sync_copy(r0, s_sh.at[pl.ds(base + i * _KE, _KE)])
    pltpu.sync_copy(r0.at[pl.ds(0, 64)], s_sh.at[pl.ds(base + 560, 64)])

    @pl.when(sid == _NS - 1)
    def _ztail():
        pltpu.sync_copy(r0.at[pl.ds(0, _REM)],
                        s_sh.at[pl.ds(_NS * _RPT, _REM)])

    if do_deg:
        def _zo(k, carry):
            ones_v[k, pl.ds(0, 16)] = jnp.zeros((16,), jnp.float32)
            return carry

        lax.fori_loop(0, _KE, _zo, 0)

        @pl.when(cid == 0)
        def _zdeg():
            for i in range(7):
                pltpu.sync_copy(ones_v,
                                deg_sh.at[pl.ds(base + i * _KE, _KE)])
            pltpu.sync_copy(ones_v.at[pl.ds(0, 64)],
                            deg_sh.at[pl.ds(base + 560, 64)])

            @pl.when(sid == _NS - 1)
            def _zdtail():
                pltpu.sync_copy(ones_v.at[pl.ds(0, _REM)],
                                deg_sh.at[pl.ds(_NS * _RPT, _REM)])

        def _ofill(k, carry):
            ones_v[k, pl.ds(0, 16)] = jnp.full((16,), 1.0, jnp.float32)
            return carry

        lax.fori_loop(0, _KE, _ofill, 0)

    plsc.subcore_barrier()

    def _start_c(g, j, p):
        ebase = sid * _EPT + g * (_GRP * _KE) + j * _KE
        pltpu.async_copy(c_hbm.at[cid, pl.ds(ebase, _KE)], a_b[p], sem_c[p])

    def _start_a(j, p):
        pltpu.async_copy(a_hbm.at[idx_dg.at[j]], a_b[p], sem_a[p], add=True)

    def _start_b(j, p):
        pltpu.async_copy(b_hbm.at[idx_s.at[j]], b_b[p], sem_b[p])

    def _wait_a(p):
        pltpu.make_async_copy(a_hbm.at[idx_dg.at[0]], a_b[p], sem_a[p]).wait()

    def _wait_b(p):
        pltpu.make_async_copy(b_hbm.at[idx_s.at[0]], b_b[p], sem_b[p]).wait()

    def _wait_c(p):
        pltpu.make_async_copy(c_hbm.at[cid, pl.ds(0, _KE)], a_b[p],
                              sem_c[p]).wait()

    def _wait_s(p):
        pltpu.make_async_copy(r_b[p], s_sh.at[idx_d.at[0]], sem_s[p]).wait()

    def _compute(j, p):
        def _row(k, inner):
            for cc in range(_HC // 16):
                sl = pl.ds(cc * 16, 16)
                r_b[p][k, sl] = jnp.maximum(a_b[p][k, sl] + b_b[p][k, sl],
                                            0.0)
            return inner

        lax.fori_loop(0, _KE, _row, 0)

    for g in range(_NG):
        pltpu.sync_copy(src_hbm.at[sid, g], idx_s)
        pltpu.sync_copy(dst_hbm.at[sid, g], idx_d)

        def _bias(k, carry2):
            for cc in range(_KE // 16):
                sl = pl.ds(cc * 16, 16)
                idx_s[k, sl] = idx_s[k, sl] + row_off
                idx_dg[k, sl] = idx_d[k, sl] + row_off
            return carry2

        lax.fori_loop(0, _GRP, _bias, 0)

        _start_c(g, 0, 0)
        _start_c(g, 1, 1)
        _start_b(0, 0)
        _wait_c(0)
        _start_a(0, 0)

        def _pair(i, carry):
            for p in (0, 1):
                j = 2 * i + p
                _wait_a(p)
                _wait_b(p)

                @pl.when(i > 0)
                def _drain():
                    _wait_s(p)

                _compute(j, p)
                pltpu.async_copy(r_b[p], s_sh.at[idx_d.at[j]], sem_s[p],
                                 add=True)
                if do_deg:
                    @pl.when(cid == 0)
                    def _deg():
                        pltpu.sync_copy(ones_v, deg_sh.at[idx_d.at[j]],
                                        add=True)

                @pl.when(j + 1 < _GRP)
                def _nexta():
                    _wait_c(1 - p)
                    _start_a(j + 1, 1 - p)

                @pl.when(j + 2 < _GRP)
                def _nextc():
                    _start_c(g, j + 2, p)

                @pl.when(j + 1 < _GRP)
                def _nextb():
                    _start_b(j + 1, 1 - p)
            return carry

        lax.fori_loop(0, _GRP // 2, _pair, 0)
        _wait_s(0)
        _wait_s(1)

    plsc.subcore_barrier()

    tail0 = _NS * _RPT
    pltpu.sync_copy(s_sh.at[pl.ds(base, _RPT)],
                    out_s.at[cid, pl.ds(base, _RPT)])

    @pl.when(sid == _NS - 1)
    def _wtail():
        pltpu.sync_copy(s_sh.at[pl.ds(tail0, _REM)],
                        out_s.at[cid, pl.ds(tail0, _REM)])

    if do_deg:
        @pl.when(cid == 0)
        def _wdeg():
            pltpu.sync_copy(deg_sh.at[pl.ds(base, _RPT)],
                            out_deg.at[pl.ds(base, _RPT)])

            @pl.when(sid == _NS - 1)
            def _wdtail():
                pltpu.sync_copy(deg_sh.at[pl.ds(tail0, _REM)],
                                out_deg.at[pl.ds(tail0, _REM)])


@functools.lru_cache(maxsize=None)
def _make_sc_edge(with_deg: bool):
    mesh = plsc.VectorSubcoreMesh(
        core_axis_name="c", subcore_axis_name="s",
        num_cores=_NC, num_subcores=_NS)
    out_type = [jax.ShapeDtypeStruct((_NC, _N, _HC), jnp.float32)]
    scratch = [
        pltpu.VMEM((_GRP, _KE), jnp.int32),
        pltpu.VMEM((_GRP, _KE), jnp.int32),
        pltpu.VMEM((_GRP, _KE), jnp.int32),
    ]
    scratch += [pltpu.VMEM((_KE, _HC), jnp.float32)] * 6
    if with_deg:
        out_type.append(jax.ShapeDtypeStruct((_N, 16), jnp.float32))
        scratch.append(pltpu.VMEM((_KE, 16), jnp.float32))
    scratch.append(pltpu.VMEM_SHARED((_N, _HC), jnp.float32))
    if with_deg:
        scratch.append(pltpu.VMEM_SHARED((_N, 16), jnp.float32))
    scratch += [pltpu.SemaphoreType.DMA] * 8
    return pl.kernel(
        functools.partial(_sc_edge_body, with_deg),
        out_type=out_type if with_deg else out_type[0],
        mesh=mesh,
        scratch_types=scratch,
        compiler_params=pltpu.CompilerParams(use_tc_tiling_on_sc=False),
    )


def _emit_tables(hh, wd_ref, ws_ref, a_ref, b4_ref):
    hb = hh.astype(jnp.bfloat16)
    for c in range(_NC):
        a_ref[c] = jnp.dot(hb, wd_ref[c].astype(jnp.bfloat16),
                           preferred_element_type=jnp.float32)
        b4_ref[c] = jnp.dot(hb, ws_ref[c].astype(jnp.bfloat16),
                            preferred_element_type=jnp.float32)


def _projab_body(x_ref, w_ref, b_ref, wd_ref, ws_ref, h_ref, a_ref, b4_ref):
    hh = (jnp.dot(x_ref[...], w_ref[...],
                  preferred_element_type=jnp.float32) + b_ref[...])
    h_ref[...] = hh
    _emit_tables(hh, wd_ref, ws_ref, a_ref, b4_ref)


_TAB_OUT = [jax.ShapeDtypeStruct((_N, _H), jnp.float32),
            jax.ShapeDtypeStruct((_NC, _N, _HC), jnp.float32),
            jax.ShapeDtypeStruct((_NC, _N, _HC), jnp.float32)]
_TAB_SPECS = [pl.BlockSpec((_BN, _H), lambda i: (i, 0)),
              pl.BlockSpec((_NC, _BN, _HC), lambda i: (0, i, 0)),
              pl.BlockSpec((_NC, _BN, _HC), lambda i: (0, i, 0))]


def _projab(x, w, b, wd, ws):
    full = lambda shape: pl.BlockSpec(shape, lambda i: tuple(0 for _ in shape))
    return pl.pallas_call(
        _projab_body,
        grid=(_N // _BN,),
        in_specs=[
            pl.BlockSpec((_BN, _NF), lambda i: (i, 0)),
            full((_NF, _H)),
            full((1, _H)),
            full((_NC, _H, _HC)),
            full((_NC, _H, _HC)),
        ],
        out_specs=_TAB_SPECS,
        out_shape=_TAB_OUT,
    )(x, w, b, wd, ws)


def _cmat_body(ea_ref, w_ref, b_ref, o_ref):
    o_ref[0] = (jnp.dot(ea_ref[...].astype(jnp.bfloat16),
                        w_ref[0].astype(jnp.bfloat16),
                        preferred_element_type=jnp.float32)
                + b_ref[0])


_E8 = _E // 8
_BE8 = 400


def _cmat(ea8, wbd, b1t):
    return pl.pallas_call(
        _cmat_body,
        grid=(_NC, _E8 // _BE8),
        in_specs=[
            pl.BlockSpec((_BE8, _NF), lambda c, i: (i, 0)),
            pl.BlockSpec((1, _NF, 8 * _HC), lambda c, i: (c, 0, 0)),
            pl.BlockSpec((1, 1, 8 * _HC), lambda c, i: (c, 0, 0)),
        ],
        out_specs=pl.BlockSpec((1, _BE8, 8 * _HC), lambda c, i: (c, i, 0)),
        out_shape=jax.ShapeDtypeStruct((_NC, _E8, 8 * _HC), jnp.float32),
    )(ea8, wbd, b1t)


def _upd_core(h_ref, s2_ref, deg_ref, w2_ref, b2_ref, u1a_ref, u1b_ref,
              ub1_ref, u2_ref, ub2_ref, g_ref, bb_ref):
    hh = h_ref[...]
    s = jnp.concatenate([s2_ref[0], s2_ref[1]], axis=-1)
    deg = deg_ref[:, 0:1]
    agg = (jnp.dot(s, w2_ref[...], preferred_element_type=jnp.float32)
           + deg * b2_ref[...])
    t = jnp.maximum(
        jnp.dot(hh, u1a_ref[...], preferred_element_type=jnp.float32)
        + jnp.dot(agg, u1b_ref[...], preferred_element_type=jnp.float32)
        + ub1_ref[...], 0.0)
    u = (jnp.dot(t, u2_ref[...], preferred_element_type=jnp.float32)
         + ub2_ref[...])
    z = hh + u
    mu = jnp.mean(z, axis=-1, keepdims=True)
    zc = z - mu
    var = jnp.mean(zc * zc, axis=-1, keepdims=True)
    return zc * lax.rsqrt(var + 1e-5) * g_ref[...] + bb_ref[...]


def _upd_specs():
    full = lambda shape: pl.BlockSpec(shape, lambda i: tuple(0 for _ in shape))
    return [
        pl.BlockSpec((_BN, _H), lambda i: (i, 0)),
        pl.BlockSpec((_NC, _BN, _HC), lambda i: (0, i, 0)),
        pl.BlockSpec((_BN, 16), lambda i: (i, 0)),
        full((_H, _H)),
        full((1, _H)),
        full((_H, _H)),
        full((_H, _H)),
        full((1, _H)),
        full((_H, _H)),
        full((1, _H)),
        full((1, _H)),
        full((1, _H)),
    ]


def _upd_body(h_ref, s2_ref, deg_ref, w2_ref, b2_ref, u1a_ref, u1b_ref,
              ub1_ref, u2_ref, ub2_ref, g_ref, bb_ref, o_ref):
    o_ref[...] = _upd_core(h_ref, s2_ref, deg_ref, w2_ref, b2_ref, u1a_ref,
                           u1b_ref, ub1_ref, u2_ref, ub2_ref, g_ref, bb_ref)


def _upd(h, s2, deg, w2, b2, u1a, u1b, ub1, u2, ub2, g, bb):
    return pl.pallas_call(
        _upd_body,
        grid=(_N // _BN,),
        in_specs=_upd_specs(),
        out_specs=pl.BlockSpec((_BN, _H), lambda i: (i, 0)),
        out_shape=jax.ShapeDtypeStruct((_N, _H), jnp.float32),
    )(h, s2, deg, w2, b2, u1a, u1b, ub1, u2, ub2, g, bb)


def _updab_body(h_ref, s2_ref, deg_ref, w2_ref, b2_ref, u1a_ref, u1b_ref,
                ub1_ref, u2_ref, ub2_ref, g_ref, bb_ref, wd_ref, ws_ref,
                o_ref, a_ref, b4_ref):
    hn = _upd_core(h_ref, s2_ref, deg_ref, w2_ref, b2_ref, u1a_ref,
                   u1b_ref, ub1_ref, u2_ref, ub2_ref, g_ref, bb_ref)
    o_ref[...] = hn
    _emit_tables(hn, wd_ref, ws_ref, a_ref, b4_ref)


def _updab(h, s2, deg, w2, b2, u1a, u1b, ub1, u2, ub2, g, bb, wd, ws):
    full = lambda shape: pl.BlockSpec(shape, lambda i: tuple(0 for _ in shape))
    return pl.pallas_call(
        _updab_body,
        grid=(_N // _BN,),
        in_specs=_upd_specs() + [full((_NC, _H, _HC)), full((_NC, _H, _HC))],
        out_specs=_TAB_SPECS,
        out_shape=_TAB_OUT,
    )(h, s2, deg, w2, b2, u1a, u1b, ub1, u2, ub2, g, bb, wd, ws)


def kernel(x, edge_index, edge_attr, W_in, b_in, msg_W1, msg_b1, msg_W2,
           msg_b2, upd_W1, upd_b1, upd_W2, upd_b2, ln_g, ln_b):
    src = edge_index[0].reshape(_NS, _NG, _GRP, _KE)
    dst = edge_index[1].reshape(_NS, _NG, _GRP, _KE)

    sc_first = _make_sc_edge(True)
    sc_rest = _make_sc_edge(False)

    def _csplit(w):
        return w.reshape(w.shape[0], _NC, _HC).swapaxes(0, 1)

    ea8 = edge_attr.reshape(_E8, _NF)
    eye8 = jnp.eye(8, dtype=jnp.float32)

    wd_l = [_csplit(msg_W1[l, :_H]) for l in range(_L)]
    ws_l = [_csplit(msg_W1[l, _H:2 * _H]) for l in range(_L)]

    h, a4, b4 = _projab(x, W_in, b_in.reshape(1, _H), wd_l[0], ws_l[0])

    c_all = []
    for l in range(_L):
        w1e2 = _csplit(msg_W1[l, 2 * _H:])
        wbd = jnp.stack([jnp.kron(eye8, w1e2[c]) for c in range(_NC)])
        b12 = _csplit(msg_b1[l].reshape(1, _H))
        b1t = jnp.tile(b12, (1, 1, 8))
        c_all.append(_cmat(ea8, wbd, b1t).reshape(_NC, _E, _HC))

    deg = None
    for l in range(_L):
        a_tab = a4.reshape(_NC * _N, _HC)
        b_tab = b4.reshape(_NC * _N, _HC)
        c_e = c_all[l]
        if l == 0:
            s2, deg = sc_first(a_tab, b_tab, c_e, src, dst)
        else:
            s2 = sc_rest(a_tab, b_tab, c_e, src, dst)
        upd_args = (h, s2, deg,
                    msg_W2[l], msg_b2[l].reshape(1, _H),
                    upd_W1[l, :_H], upd_W1[l, _H:],
                    upd_b1[l].reshape(1, _H),
                    upd_W2[l], upd_b2[l].reshape(1, _H),
                    ln_g[l].reshape(1, _H), ln_b[l].reshape(1, _H))
        if l < _L - 1:
            h, a4, b4 = _updab(*upd_args, wd_l[l + 1], ws_l[l + 1])
        else:
            h = _upd(*upd_args)
    return h


# speedup vs baseline: 1.3023x; 1.3023x over previous
import functools

import jax
import jax.numpy as jnp
from jax import lax
from jax.experimental import pallas as pl
from jax.experimental.pallas import tpu as pltpu
from jax.experimental.pallas import tpu_sc as plsc

_N, _E, _NF, _EF, _H, _L = 10000, 320000, 128, 16, 128, 6
_NC, _NS = 2, 16
_HC = _H // _NC
_KE = 80
_NBT = _E // (_NS * _KE)
_GRP = 50
_NG = _NBT // _GRP
_EPT = _NBT * _KE
_RPT = 624
_REM = _N - _NS * _RPT
_ZR = 208

_BN = 400
_BE = 3200


def _sc_edge_body(with_deg, *refs):
    if with_deg:
        (a_hbm, b_hbm, c_hbm, src_hbm, dst_hbm, out_s, out_deg,
         idx_s, idx_d, idx_dg, a0, a1, b0, b1, c0, c1, r0, r1,
         ones_v, s_sh, deg_sh,
         sem_a0, sem_a1, sem_b0, sem_b1, sem_c0, sem_c1,
         sem_s0, sem_s1) = refs
    else:
        (a_hbm, b_hbm, c_hbm, src_hbm, dst_hbm, out_s,
         idx_s, idx_d, idx_dg, a0, a1, b0, b1, c0, c1, r0, r1, s_sh,
         sem_a0, sem_a1, sem_b0, sem_b1, sem_c0, sem_c1,
         sem_s0, sem_s1) = refs

    a_b = (a0, a1)
    b_b = (b0, b1)
    c_b = (c0, c1)
    r_b = (r0, r1)
    sem_a = (sem_a0, sem_a1)
    sem_b = (sem_b0, sem_b1)
    sem_c = (sem_c0, sem_c1)
    sem_s = (sem_s0, sem_s1)

    cid = lax.axis_index("c")
    sid = lax.axis_index("s")
    do_deg = with_deg
    row_off = cid * _N

    def _zrow(k, carry):
        for cc in range(_HC // 16):
            sl = pl.ds(cc * 16, 16)
            r0[k, sl] = jnp.zeros((16,), jnp.float32)
        return carry

    lax.fori_loop(0, _KE, _zrow, 0)
    base = sid * _RPT
    for i in range(7):
        pltpu.sync_copy(r0, s_sh.at[pl.ds(base + i * _KE, _KE)])
    pltpu.sync_copy(r0.at[pl.ds(0, 64)], s_sh.at[pl.ds(base + 560, 64)])

    @pl.when(sid == _NS - 1)
    def _ztail():
        pltpu.sync_copy(r0.at[pl.ds(0, _REM)],
                        s_sh.at[pl.ds(_NS * _RPT, _REM)])

    if do_deg:
        def _zo(k, carry):
            ones_v[k, pl.ds(0, 16)] = jnp.zeros((16,), jnp.float32)
            return carry

        lax.fori_loop(0, _KE, _zo, 0)

        @pl.when(cid == 0)
        def _zdeg():
            for i in range(7):
                pltpu.sync_copy(ones_v,
                                deg_sh.at[pl.ds(base + i * _KE, _KE)])
            pltpu.sync_copy(ones_v.at[pl.ds(0, 64)],
                            deg_sh.at[pl.ds(base + 560, 64)])

            @pl.when(sid == _NS - 1)
            def _zdtail():
                pltpu.sync_copy(ones_v.at[pl.ds(0, _REM)],
                                deg_sh.at[pl.ds(_NS * _RPT, _REM)])

        def _ofill(k, carry):
            ones_v[k, pl.ds(0, 16)] = jnp.full((16,), 1.0, jnp.float32)
            return carry

        lax.fori_loop(0, _KE, _ofill, 0)

    plsc.subcore_barrier()

    def _start3(g, j, p):
        ebase = sid * _EPT + g * (_GRP * _KE) + j * _KE
        pltpu.async_copy(a_hbm.at[idx_dg.at[j]], a_b[p], sem_a[p])
        pltpu.async_copy(b_hbm.at[idx_s.at[j]], b_b[p], sem_b[p])
        pltpu.async_copy(c_hbm.at[cid, pl.ds(ebase, _KE)], c_b[p], sem_c[p])

    def _wait3(p):
        pltpu.make_async_copy(a_hbm.at[idx_dg.at[0]], a_b[p], sem_a[p]).wait()
        pltpu.make_async_copy(b_hbm.at[idx_s.at[0]], b_b[p], sem_b[p]).wait()
        pltpu.make_async_copy(c_hbm.at[cid, pl.ds(0, _KE)], c_b[p],
                              sem_c[p]).wait()

    def _wait_s(p):
        pltpu.make_async_copy(r_b[p], s_sh.at[idx_d.at[0]], sem_s[p]).wait()

    def _compute(j, p):
        def _row(k, inner):
            for cc in range(_HC // 16):
                sl = pl.ds(cc * 16, 16)
                r_b[p][k, sl] = jnp.maximum(
                    a_b[p][k, sl] + b_b[p][k, sl] + c_b[p][k, sl], 0.0)
            return inner

        lax.fori_loop(0, _KE, _row, 0)

    for g in range(_NG):
        pltpu.sync_copy(src_hbm.at[sid, g], idx_s)
        pltpu.sync_copy(dst_hbm.at[sid, g], idx_d)

        def _bias(k, carry2):
            for cc in range(_KE // 16):
                sl = pl.ds(cc * 16, 16)
                idx_s[k, sl] = idx_s[k, sl] + row_off
                idx_dg[k, sl] = idx_d[k, sl] + row_off
            return carry2

        lax.fori_loop(0, _GRP, _bias, 0)

        _start3(g, 0, 0)
        _start3(g, 1, 1)

        def _pair(i, carry):
            for p in (0, 1):
                j = 2 * i + p
                _wait3(p)

                @pl.when(i > 0)
                def _drain():
                    _wait_s(p)

                _compute(j, p)

                @pl.when(i < _GRP // 2 - 1)
                def _more():
                    _start3(g, j + 2, p)

                pltpu.async_copy(r_b[p], s_sh.at[idx_d.at[j]], sem_s[p],
                                 add=True)
                if do_deg:
                    @pl.when(cid == 0)
                    def _deg():
                        pltpu.sync_copy(ones_v, deg_sh.at[idx_d.at[j]],
                                        add=True)
            return carry

        lax.fori_loop(0, _GRP // 2, _pair, 0)
        _wait_s(0)
        _wait_s(1)

    plsc.subcore_barrier()

    tail0 = _NS * _RPT
    pltpu.sync_copy(s_sh.at[pl.ds(base, _RPT)],
                    out_s.at[cid, pl.ds(base, _RPT)])

    @pl.when(sid == _NS - 1)
    def _wtail():
        pltpu.sync_copy(s_sh.at[pl.ds(tail0, _REM)],
                        out_s.at[cid, pl.ds(tail0, _REM)])

    if do_deg:
        @pl.when(cid == 0)
        def _wdeg():
            pltpu.sync_copy(deg_sh.at[pl.ds(base, _RPT)],
                            out_deg.at[pl.ds(base, _RPT)])

            @pl.when(sid == _NS - 1)
            def _wdtail():
                pltpu.sync_copy(deg_sh.at[pl.ds(tail0, _REM)],
                                out_deg.at[pl.ds(tail0, _REM)])


@functools.lru_cache(maxsize=None)
def _make_sc_edge(with_deg: bool):
    mesh = plsc.VectorSubcoreMesh(
        core_axis_name="c", subcore_axis_name="s",
        num_cores=_NC, num_subcores=_NS)
    out_type = [jax.ShapeDtypeStruct((_NC, _N, _HC), jnp.float32)]
    scratch = [
        pltpu.VMEM((_GRP, _KE), jnp.int32),
        pltpu.VMEM((_GRP, _KE), jnp.int32),
        pltpu.VMEM((_GRP, _KE), jnp.int32),
    ]
    scratch += [pltpu.VMEM((_KE, _HC), jnp.float32)] * 8
    if with_deg:
        out_type.append(jax.ShapeDtypeStruct((_N, 16), jnp.float32))
        scratch.append(pltpu.VMEM((_KE, 16), jnp.float32))
    scratch.append(pltpu.VMEM_SHARED((_N, _HC), jnp.float32))
    if with_deg:
        scratch.append(pltpu.VMEM_SHARED((_N, 16), jnp.float32))
    scratch += [pltpu.SemaphoreType.DMA] * 8
    return pl.kernel(
        functools.partial(_sc_edge_body, with_deg),
        out_type=out_type if with_deg else out_type[0],
        mesh=mesh,
        scratch_types=scratch,
        compiler_params=pltpu.CompilerParams(use_tc_tiling_on_sc=False),
    )


def _emit_tables(hh, wd_ref, ws_ref, a_ref, b4_ref):
    hb = hh.astype(jnp.bfloat16)
    for c in range(_NC):
        a_ref[c] = jnp.dot(hb, wd_ref[c].astype(jnp.bfloat16),
                           preferred_element_type=jnp.float32)
        b4_ref[c] = jnp.dot(hb, ws_ref[c].astype(jnp.bfloat16),
                            preferred_element_type=jnp.float32)


def _projab_body(x_ref, w_ref, b_ref, wd_ref, ws_ref, h_ref, a_ref, b4_ref):
    hh = (jnp.dot(x_ref[...], w_ref[...],
                  preferred_element_type=jnp.float32) + b_ref[...])
    h_ref[...] = hh
    _emit_tables(hh, wd_ref, ws_ref, a_ref, b4_ref)


_TAB_OUT = [jax.ShapeDtypeStruct((_N, _H), jnp.float32),
            jax.ShapeDtypeStruct((_NC, _N, _HC), jnp.float32),
            jax.ShapeDtypeStruct((_NC, _N, _HC), jnp.float32)]
_TAB_SPECS = [pl.BlockSpec((_BN, _H), lambda i: (i, 0)),
              pl.BlockSpec((_NC, _BN, _HC), lambda i: (0, i, 0)),
              pl.BlockSpec((_NC, _BN, _HC), lambda i: (0, i, 0))]


def _projab(x, w, b, wd, ws):
    full = lambda shape: pl.BlockSpec(shape, lambda i: tuple(0 for _ in shape))
    return pl.pallas_call(
        _projab_body,
        grid=(_N // _BN,),
        in_specs=[
            pl.BlockSpec((_BN, _NF), lambda i: (i, 0)),
            full((_NF, _H)),
            full((1, _H)),
            full((_NC, _H, _HC)),
            full((_NC, _H, _HC)),
        ],
        out_specs=_TAB_SPECS,
        out_shape=_TAB_OUT,
    )(x, w, b, wd, ws)


def _cmat_body(ea_ref, w_ref, b_ref, o_ref):
    o_ref[0] = (jnp.dot(ea_ref[...].astype(jnp.bfloat16),
                        w_ref[0].astype(jnp.bfloat16),
                        preferred_element_type=jnp.float32)
                + b_ref[0])


_E8 = _E // 8
_BE8 = 400


def _cmat(ea8, wbd, b1t):
    return pl.pallas_call(
        _cmat_body,
        grid=(_NC, _E8 // _BE8),
        in_specs=[
            pl.BlockSpec((_BE8, _NF), lambda c, i: (i, 0)),
            pl.BlockSpec((1, _NF, 8 * _HC), lambda c, i: (c, 0, 0)),
            pl.BlockSpec((1, 1, 8 * _HC), lambda c, i: (c, 0, 0)),
        ],
        out_specs=pl.BlockSpec((1, _BE8, 8 * _HC), lambda c, i: (c, i, 0)),
        out_shape=jax.ShapeDtypeStruct((_NC, _E8, 8 * _HC), jnp.float32),
    )(ea8, wbd, b1t)


def _upd_core(h_ref, s2_ref, deg_ref, w2_ref, b2_ref, u1a_ref, u1b_ref,
              ub1_ref, u2_ref, ub2_ref, g_ref, bb_ref):
    hh = h_ref[...]
    s = jnp.concatenate([s2_ref[0], s2_ref[1]], axis=-1)
    deg = deg_ref[:, 0:1]
    agg = (jnp.dot(s, w2_ref[...], preferred_element_type=jnp.float32)
           + deg * b2_ref[...])
    t = jnp.maximum(
        jnp.dot(hh, u1a_ref[...], preferred_element_type=jnp.float32)
        + jnp.dot(agg, u1b_ref[...], preferred_element_type=jnp.float32)
        + ub1_ref[...], 0.0)
    u = (jnp.dot(t, u2_ref[...], preferred_element_type=jnp.float32)
         + ub2_ref[...])
    z = hh + u
    mu = jnp.mean(z, axis=-1, keepdims=True)
    zc = z - mu
    var = jnp.mean(zc * zc, axis=-1, keepdims=True)
    return zc * lax.rsqrt(var + 1e-5) * g_ref[...] + bb_ref[...]


def _upd_specs():
    full = lambda shape: pl.BlockSpec(shape, lambda i: tuple(0 for _ in shape))
    return [
        pl.BlockSpec((_BN, _H), lambda i: (i, 0)),
        pl.BlockSpec((_NC, _BN, _HC), lambda i: (0, i, 0)),
        pl.BlockSpec((_BN, 16), lambda i: (i, 0)),
        full((_H, _H)),
        full((1, _H)),
        full((_H, _H)),
        full((_H, _H)),
        full((1, _H)),
        full((_H, _H)),
        full((1, _H)),
        full((1, _H)),
        full((1, _H)),
    ]


def _upd_body(h_ref, s2_ref, deg_ref, w2_ref, b2_ref, u1a_ref, u1b_ref,
              ub1_ref, u2_ref, ub2_ref, g_ref, bb_ref, o_ref):
    o_ref[...] = _upd_core(h_ref, s2_ref, deg_ref, w2_ref, b2_ref, u1a_ref,
                           u1b_ref, ub1_ref, u2_ref, ub2_ref, g_ref, bb_ref)


def _upd(h, s2, deg, w2, b2, u1a, u1b, ub1, u2, ub2, g, bb):
    return pl.pallas_call(
        _upd_body,
        grid=(_N // _BN,),
        in_specs=_upd_specs(),
        out_specs=pl.BlockSpec((_BN, _H), lambda i: (i, 0)),
        out_shape=jax.ShapeDtypeStruct((_N, _H), jnp.float32),
    )(h, s2, deg, w2, b2, u1a, u1b, ub1, u2, ub2, g, bb)


def _updab_body(h_ref, s2_ref, deg_ref, w2_ref, b2_ref, u1a_ref, u1b_ref,
                ub1_ref, u2_ref, ub2_ref, g_ref, bb_ref, wd_ref, ws_ref,
                o_ref, a_ref, b4_ref):
    hn = _upd_core(h_ref, s2_ref, deg_ref, w2_ref, b2_ref, u1a_ref,
                   u1b_ref, ub1_ref, u2_ref, ub2_ref, g_ref, bb_ref)
    o_ref[...] = hn
    _emit_tables(hn, wd_ref, ws_ref, a_ref, b4_ref)


def _updab(h, s2, deg, w2, b2, u1a, u1b, ub1, u2, ub2, g, bb, wd, ws):
    full = lambda shape: pl.BlockSpec(shape, lambda i: tuple(0 for _ in shape))
    return pl.pallas_call(
        _updab_body,
        grid=(_N // _BN,),
        in_specs=_upd_specs() + [full((_NC, _H, _HC)), full((_NC, _H, _HC))],
        out_specs=_TAB_SPECS,
        out_shape=_TAB_OUT,
    )(h, s2, deg, w2, b2, u1a, u1b, ub1, u2, ub2, g, bb, wd, ws)


def kernel(x, edge_index, edge_attr, W_in, b_in, msg_W1, msg_b1, msg_W2,
           msg_b2, upd_W1, upd_b1, upd_W2, upd_b2, ln_g, ln_b):
    src = edge_index[0].reshape(_NS, _NG, _GRP, _KE)
    dst = edge_index[1].reshape(_NS, _NG, _GRP, _KE)

    sc_first = _make_sc_edge(True)
    sc_rest = _make_sc_edge(False)

    def _csplit(w):
        return w.reshape(w.shape[0], _NC, _HC).swapaxes(0, 1)

    ea8 = edge_attr.reshape(_E8, _NF)
    eye8 = jnp.eye(8, dtype=jnp.float32)

    wd_l = [_csplit(msg_W1[l, :_H]) for l in range(_L)]
    ws_l = [_csplit(msg_W1[l, _H:2 * _H]) for l in range(_L)]

    h, a4, b4 = _projab(x, W_in, b_in.reshape(1, _H), wd_l[0], ws_l[0])

    c_all = []
    for l in range(_L):
        w1e2 = _csplit(msg_W1[l, 2 * _H:])
        wbd = jnp.stack([jnp.kron(eye8, w1e2[c]) for c in range(_NC)])
        b12 = _csplit(msg_b1[l].reshape(1, _H))
        b1t = jnp.tile(b12, (1, 1, 8))
        c_all.append(_cmat(ea8, wbd, b1t).reshape(_NC, _E, _HC))

    deg = None
    for l in range(_L):
        a_tab = a4.reshape(_NC * _N, _HC)
        b_tab = b4.reshape(_NC * _N, _HC)
        c_e = c_all[l]
        if l == 0:
            s2, deg = sc_first(a_tab, b_tab, c_e, src, dst)
        else:
            s2 = sc_rest(a_tab, b_tab, c_e, src, dst)
        upd_args = (h, s2, deg,
                    msg_W2[l], msg_b2[l].reshape(1, _H),
                    upd_W1[l, :_H], upd_W1[l, _H:],
                    upd_b1[l].reshape(1, _H),
                    upd_W2[l], upd_b2[l].reshape(1, _H),
                    ln_g[l].reshape(1, _H), ln_b[l].reshape(1, _H))
        if l < _L - 1:
            h, a4, b4 = _updab(*upd_args, wd_l[l + 1], ws_l[l + 1])
        else:
            h = _upd(*upd_args)
    return h
